# Initial kernel scaffold; baseline (speedup 1.0000x reference)
import jax
import jax.numpy as jnp
from jax.experimental import pallas as pl


def kernel(x, edge_index, W1, b1, W2, b2, W3, b3, W4, b4):
    raise NotImplementedError("write your pallas kernel here")


# speedup vs baseline: 1.0374x; 1.0374x over previous
import functools

import jax
import jax.numpy as jnp
from jax import lax
from jax.experimental import pallas as pl
from jax.experimental.pallas import tpu as pltpu
from jax.experimental.pallas import tpu_sc as plsc

N = 10000
E = 160000
NPAD = 10240
R = 160
NB = 63
NOUT = NB * R
NC = 2
NS = 16
NW = NC * NS
CH = 2000
NCH = E // CH
NV = CH // 16
SB = 32

_mesh = lambda: plsc.VectorSubcoreMesh(
    core_axis_name="c", subcore_axis_name="s", num_cores=NC, num_subcores=NS)


RD = NPAD // NW


def _deg_body(dst_hbm, out_hbm, dstb, cntf, cnt):
    wid = lax.axis_index("s") * NC + lax.axis_index("c")
    lo = wid * RD

    zf = jnp.zeros((16,), jnp.float32)
    ones = jnp.ones((16,), jnp.float32)
    lanes = lax.iota(jnp.int32, 16)

    def zero(i, _):
        cntf[pl.ds(i * 16, 16)] = zf
        return 0

    lax.fori_loop(0, RD, zero, 0)

    def chunk(ci, _):
        pltpu.sync_copy(dst_hbm.at[pl.ds(ci * CH, CH)], dstb)

        def scan(v, _):
            dv = dstb[pl.ds(v * 16, 16)]
            ld = dv - lo
            m = (ld >= 0) & (ld < RD)
            plsc.addupdate_scatter(cntf, [lanes * RD + ld], ones, mask=m)
            return 0

        lax.fori_loop(0, NV, scan, 0)
        return 0

    lax.fori_loop(0, NCH, chunk, 0)

    def red(g, _):
        s = cntf[pl.ds(g * 16, 16)]
        for l in range(1, 16):
            s = s + cntf[pl.ds(l * RD + g * 16, 16)]
        cnt[pl.ds(g * 16, 16)] = s
        return 0

    lax.fori_loop(0, RD // 16, red, 0)
    pltpu.sync_copy(cnt, out_hbm.at[pl.ds(lo, RD)])


def _sc_degree(dst):
    f = pl.kernel(
        _deg_body,
        out_type=jax.ShapeDtypeStruct((NPAD,), jnp.float32),
        mesh=_mesh(),
        compiler_params=pltpu.CompilerParams(needs_layout_passes=False),
        scratch_types=[
            pltpu.VMEM((CH,), jnp.int32),
            pltpu.VMEM((16 * RD,), jnp.float32),
            pltpu.VMEM((RD,), jnp.float32),
        ],
    )
    return f(dst)


def _prop_body(p_hbm, src_hbm, dst_hbm, dinv_hbm, out_hbm,
               acc, dinv_v, srcb, dstb, src_c, ldst_c, idxb, rows,
               sem, D):
    wid = lax.axis_index("s") * NC + lax.axis_index("c")
    pltpu.sync_copy(dinv_hbm, dinv_v)

    zf = jnp.zeros((16,), jnp.float32)
    zi = jnp.zeros((16,), jnp.int32)
    DK = D // 16

    def one_pass(p, _):
        blk = wid + p * NW

        @pl.when(blk < NB)
        def _():
            lo = blk * R

            def zero(i, _):
                r = i // DK
                k = i - r * DK
                acc[r, pl.ds(k * 16, 16)] = zf
                return 0

            lax.fori_loop(0, R * DK, zero, 0)

            def chunk_body(ci, _):
                pltpu.sync_copy(src_hbm.at[pl.ds(ci * CH, CH)], srcb)
                pltpu.sync_copy(dst_hbm.at[pl.ds(ci * CH, CH)], dstb)

                def zc(i, _):
                    src_c[pl.ds(i * 16, 16)] = zi
                    ldst_c[pl.ds(i * 16, 16)] = zi
                    return 0

                lax.fori_loop(0, (CH + 16) // 16, zc, 0)

                def scan(v, off):
                    dv = dstb[pl.ds(v * 16, 16)]
                    sv = srcb[pl.ds(v * 16, 16)]
                    ld = dv - lo
                    m = (ld >= 0) & (ld < R)
                    mi = m.astype(jnp.int32)
                    pos = plsc.cumsum(mi) + (off - 1)
                    plsc.store_scatter(src_c, [pos], sv, mask=m)
                    plsc.store_scatter(ldst_c, [pos], ld, mask=m)
                    return off + jnp.sum(mi)

                off = lax.fori_loop(0, NV, scan, 0)
                nsb = (off + SB - 1) // SB

                def sub_batch(s, _):
                    base = pl.multiple_of(s * SB, SB)
                    for q in range(SB // 16):
                        idxb[pl.ds(q * 16, 16)] = src_c[pl.ds(base + q * 16, 16)]
                    pltpu.async_copy(p_hbm.at[idxb], rows, sem).wait()
                    for q in range(SB // 16):
                        sv = idxb[pl.ds(q * 16, 16)]
                        lv = ldst_c[pl.ds(base + q * 16, 16)]
                        nv = (plsc.load_gather(dinv_v, [sv])
                              * plsc.load_gather(dinv_v, [lv + lo]))
                        for j in range(16):
                            @pl.when(base + (q * 16 + j) < off)
                            def __(q=q, j=j, lv=lv, nv=nv):
                                ldr = lv[j]
                                nr = nv[j]
                                for k in range(DK):
                                    plsc.addupdate(
                                        acc.at[ldr, pl.ds(k * 16, 16)],
                                        rows[q * 16 + j, pl.ds(k * 16, 16)] * nr)
                    return 0

                lax.fori_loop(0, nsb, sub_batch, 0)
                return 0

            lax.fori_loop(0, NCH, chunk_body, 0)
            pltpu.sync_copy(acc, out_hbm.at[pl.ds(blk * R, R)])

        return 0

    lax.fori_loop(0, 2, one_pass, 0)


def _make_propagate(D):
    body = functools.partial(_prop_body, D=D)
    return pl.kernel(
        body,
        out_type=jax.ShapeDtypeStruct((NOUT, D), jnp.float32),
        mesh=_mesh(),
        compiler_params=pltpu.CompilerParams(needs_layout_passes=False),
        scratch_types=[
            pltpu.VMEM((R, D), jnp.float32),
            pltpu.VMEM((NPAD,), jnp.float32),
            pltpu.VMEM((CH,), jnp.int32),
            pltpu.VMEM((CH,), jnp.int32),
            pltpu.VMEM((CH + 16,), jnp.int32),
            pltpu.VMEM((CH + 16,), jnp.int32),
            pltpu.VMEM((SB,), jnp.int32),
            pltpu.VMEM((SB, D), jnp.float32),
            pltpu.SemaphoreType.DMA,
        ],
    )


_propagate = {D: _make_propagate(D) for D in (512, 256)}


def _norm_body(deg_ref, dinv_ref, dinv2_ref):
    deg = deg_ref[...] + 1.0
    dinv_ref[...] = lax.rsqrt(deg)
    dinv2_ref[...] = 1.0 / deg


def _tc_norm(deg):
    return pl.pallas_call(
        _norm_body,
        out_shape=(jax.ShapeDtypeStruct((NPAD,), jnp.float32),
                   jax.ShapeDtypeStruct((NPAD,), jnp.float32)),
    )(deg)


def _mm_body(x_ref, w_ref, o_ref):
    o_ref[...] = jnp.dot(x_ref[...], w_ref[...],
                         preferred_element_type=jnp.float32)


def _tc_matmul(x, W, rows_blk=1000):
    n, din = x.shape
    dout = W.shape[1]
    grid = (n // rows_blk,)
    return pl.pallas_call(
        _mm_body,
        grid=grid,
        in_specs=[
            pl.BlockSpec((rows_blk, din), lambda i: (i, 0)),
            pl.BlockSpec((din, dout), lambda i: (0, 0)),
        ],
        out_specs=pl.BlockSpec((rows_blk, dout), lambda i: (i, 0)),
        out_shape=jax.ShapeDtypeStruct((n, dout), jnp.float32),
    )(x, W)


def _fused_body(s_ref, p_ref, d2_ref, b_ref, w_ref, o_ref):
    h = s_ref[...] + p_ref[...] * d2_ref[...] + b_ref[...]
    h = jnp.maximum(h, 0.0)
    o_ref[...] = jnp.dot(h, w_ref[...], preferred_element_type=jnp.float32)


def _tc_fused_matmul(S, P, d2col, brow, W, rows_blk=1000):
    n, din = P.shape
    dout = W.shape[1]
    grid = (n // rows_blk,)
    return pl.pallas_call(
        _fused_body,
        grid=grid,
        in_specs=[
            pl.BlockSpec((rows_blk, din), lambda i: (i, 0)),
            pl.BlockSpec((rows_blk, din), lambda i: (i, 0)),
            pl.BlockSpec((rows_blk, 1), lambda i: (i, 0)),
            pl.BlockSpec((1, din), lambda i: (0, 0)),
            pl.BlockSpec((din, dout), lambda i: (0, 0)),
        ],
        out_specs=pl.BlockSpec((rows_blk, dout), lambda i: (i, 0)),
        out_shape=jax.ShapeDtypeStruct((n, dout), jnp.float32),
    )(S, P, d2col, brow, W)


def _final_body(s_ref, p_ref, d2_ref, b_ref, o_ref):
    z = s_ref[...] + p_ref[...] * d2_ref[...] + b_ref[...]
    m = jnp.max(z, axis=1, keepdims=True)
    lse = jnp.log(jnp.sum(jnp.exp(z - m), axis=1, keepdims=True)) + m
    o_ref[...] = z - lse


def _tc_final(S, P, d2col, brow, rows_blk=1000):
    n, d = P.shape
    grid = (n // rows_blk,)
    return pl.pallas_call(
        _final_body,
        grid=grid,
        in_specs=[
            pl.BlockSpec((rows_blk, d), lambda i: (i, 0)),
            pl.BlockSpec((rows_blk, d), lambda i: (i, 0)),
            pl.BlockSpec((rows_blk, 1), lambda i: (i, 0)),
            pl.BlockSpec((1, d), lambda i: (0, 0)),
        ],
        out_specs=pl.BlockSpec((rows_blk, d), lambda i: (i, 0)),
        out_shape=jax.ShapeDtypeStruct((n, d), jnp.float32),
    )(S, P, d2col, brow)


def kernel(x, edge_index, W1, b1, W2, b2, W3, b3, W4, b4):
    src = edge_index[0]
    dst = edge_index[1]

    deg = _sc_degree(dst)
    dinv, dinv2 = _tc_norm(deg)
    d2col = dinv2[:N, None]

    P1 = _tc_matmul(x, W1)
    S1 = _propagate[512](P1, src, dst, dinv)
    P2 = _tc_fused_matmul(S1, P1, d2col, b1[None, :], W2)
    S2 = _propagate[512](P2, src, dst, dinv)
    P3 = _tc_fused_matmul(S2, P2, d2col, b2[None, :], W3)
    S3 = _propagate[512](P3, src, dst, dinv)
    P4 = _tc_fused_matmul(S3, P3, d2col, b3[None, :], W4)
    S4 = _propagate[256](P4, src, dst, dinv)
    return _tc_final(S4, P4, d2col, b4[None, :])


# speedup vs baseline: 14.1075x; 13.5990x over previous
import functools

import jax
import jax.numpy as jnp
from jax import lax
from jax.experimental import pallas as pl
from jax.experimental.pallas import tpu as pltpu
from jax.experimental.pallas import tpu_sc as plsc

N = 10000
E = 160000
NPAD = 10240
NC = 2
NS = 16
NW = NC * NS
CH = 2000
NCH = E // CH
NV = CH // 16
QW = 128
BT = 80
EPT16 = E // NS
NBT = EPT16 // BT
RPS = NPAD // NS
BZ = 40

_mesh = lambda: plsc.VectorSubcoreMesh(
    core_axis_name="c", subcore_axis_name="s", num_cores=NC, num_subcores=NS)


NEQ = 4
NRG = NW // NEQ
RD = NPAD // NRG
EQ = E // NEQ
DCH = EQ // 2


def _deg_body(dst_hbm, out_hbm, dstb0, dstb1, cntf, cnt, sem0, sem1):
    wid = lax.axis_index("s") * NC + lax.axis_index("c")
    h = wid // NRG
    r = wid - h * NRG
    lo = r * RD
    e0 = h * EQ

    zf = jnp.zeros((16,), jnp.float32)
    ones = jnp.ones((16,), jnp.float32)
    lanes = lax.iota(jnp.int32, 16)

    bufs = (dstb0, dstb1)
    sems = (sem0, sem1)
    pltpu.async_copy(dst_hbm.at[pl.ds(e0, DCH)], dstb0, sem0)

    def zero(i, _):
        cntf[pl.ds(i * 16, 16)] = zf
        return 0

    lax.fori_loop(0, RD, zero, 0)

    def scan(dstb):
        def body(v, _):
            dv = dstb[pl.ds(v * 16, 16)]
            ld = dv - lo
            m = (ld >= 0) & (ld < RD)
            plsc.addupdate_scatter(cntf, [lanes * RD + ld], ones, mask=m)
            return 0
        lax.fori_loop(0, DCH // 16, body, 0, unroll=8)

    for c in range(2):
        if c + 1 < 2:
            pltpu.async_copy(dst_hbm.at[pl.ds(e0 + (c + 1) * DCH, DCH)],
                             bufs[(c + 1) % 2], sems[(c + 1) % 2])
        pltpu.make_async_copy(dst_hbm.at[pl.ds(e0 + c * DCH, DCH)],
                              bufs[c % 2], sems[c % 2]).wait()
        scan(bufs[c % 2])

    def red(g, _):
        t = cntf[pl.ds(g * 16, 16)]
        for l in range(1, 16):
            t = t + cntf[pl.ds(l * RD + g * 16, 16)]
        cnt[pl.ds(g * 16, 16)] = t
        return 0

    lax.fori_loop(0, RD // 16, red, 0)
    pltpu.sync_copy(cnt, out_hbm.at[h, pl.ds(lo, RD)])


def _sc_degree(dst):
    f = pl.kernel(
        _deg_body,
        out_type=jax.ShapeDtypeStruct((NEQ, NPAD), jnp.float32),
        mesh=_mesh(),
        compiler_params=pltpu.CompilerParams(needs_layout_passes=False),
        scratch_types=[
            pltpu.VMEM((DCH,), jnp.int32),
            pltpu.VMEM((DCH,), jnp.int32),
            pltpu.VMEM((16 * RD,), jnp.float32),
            pltpu.VMEM((RD,), jnp.float32),
            pltpu.SemaphoreType.DMA,
            pltpu.SemaphoreType.DMA,
        ],
    )
    return f(dst)


def _prop_body(p_hbm, src_hbm, dst_hbm, out_hbm,
               srcb, dstb, idxs0, idxs1, idxd, rows0, rows1, zbuf, shared,
               semg0, semg1, D):
    cid = lax.axis_index("c")
    sid = lax.axis_index("s")
    qpc = D // QW // NC

    zf = jnp.zeros((16,), jnp.float32)

    def zzero(i, _):
        r = i // (QW // 16)
        k = i - r * (QW // 16)
        zbuf[r, pl.ds(k * 16, 16)] = zf
        return 0

    base_e = sid * EPT16
    cps = pltpu.async_copy(src_hbm.at[pl.ds(base_e, EPT16)], srcb, semg0)
    cpd = pltpu.async_copy(dst_hbm.at[pl.ds(base_e, EPT16)], dstb, semg1)
    lax.fori_loop(0, BZ * (QW // 16), zzero, 0)
    cps.wait()
    cpd.wait()

    for qq in range(qpc):
        q = cid * qpc + qq
        c0 = pl.multiple_of(q * QW, QW)

        zcps = [pltpu.async_copy(
                    zbuf, shared.at[pl.ds(sid * RPS + z * BZ, BZ)], semg0)
                for z in range(RPS // BZ)]
        for cp in zcps:
            cp.wait()
        plsc.subcore_barrier()

        def start_gather(b, idxs, rows, semg):
            for g in range(BT // 16):
                idxs[pl.ds(g * 16, 16)] = srcb[pl.ds(b * BT + g * 16, 16)]
            return pltpu.async_copy(p_hbm.at[idxs, pl.ds(c0, QW)], rows, semg)

        start_gather(0, idxs0, rows0, semg0)

        def half(b, idxs_n, rows_n, semg_n, rows_c, semg_c):
            @pl.when(b < NBT)
            def _():
                @pl.when(b + 1 < NBT)
                def _():
                    start_gather(b + 1, idxs_n, rows_n, semg_n)
                pltpu.make_async_copy(
                    p_hbm.at[idxs_n, pl.ds(c0, QW)], rows_c, semg_c).wait()
                for g in range(BT // 16):
                    idxd[pl.ds(g * 16, 16)] = dstb[pl.ds(b * BT + g * 16, 16)]
                pltpu.sync_copy(rows_c, shared.at[idxd], add=True)
            return 0

        def pair(i, _):
            b = i * 2
            half(b, idxs1, rows1, semg1, rows0, semg0)
            half(b + 1, idxs0, rows0, semg0, rows1, semg1)
            return 0

        lax.fori_loop(0, (NBT + 1) // 2, pair, 0)
        plsc.subcore_barrier()
        pltpu.sync_copy(shared.at[pl.ds(sid * RPS, RPS)],
                        out_hbm.at[pl.ds(sid * RPS, RPS), pl.ds(c0, QW)])
        plsc.subcore_barrier()


def _make_propagate(D):
    body = functools.partial(_prop_body, D=D)
    return pl.kernel(
        body,
        out_type=jax.ShapeDtypeStruct((NPAD, D), jnp.float32),
        mesh=_mesh(),
        compiler_params=pltpu.CompilerParams(needs_layout_passes=False),
        scratch_types=[
            pltpu.VMEM((EPT16,), jnp.int32),
            pltpu.VMEM((EPT16,), jnp.int32),
            pltpu.VMEM((BT,), jnp.int32),
            pltpu.VMEM((BT,), jnp.int32),
            pltpu.VMEM((BT,), jnp.int32),
            pltpu.VMEM((BT, QW), jnp.float32),
            pltpu.VMEM((BT, QW), jnp.float32),
            pltpu.VMEM((BZ, QW), jnp.float32),
            pltpu.VMEM_SHARED((NPAD, QW), jnp.float32),
            pltpu.SemaphoreType.DMA,
            pltpu.SemaphoreType.DMA,
        ],
    )


_propagate = {D: _make_propagate(D) for D in (512, 256)}


def _mm_body(x_ref, w_ref, deg_ref, o_ref):
    d = lax.rsqrt(deg_ref[...] + 1.0)
    o_ref[...] = d * jnp.dot(x_ref[...], w_ref[...],
                             preferred_element_type=jnp.float32)


def _tc_matmul(x, W, degcol, rows_blk=2000):
    n, din = x.shape
    dout = W.shape[1]
    return pl.pallas_call(
        _mm_body,
        grid=(n // rows_blk,),
        in_specs=[
            pl.BlockSpec((rows_blk, din), lambda i: (i, 0)),
            pl.BlockSpec((din, dout), lambda i: (0, 0)),
            pl.BlockSpec((rows_blk, 1), lambda i: (i, 0)),
        ],
        out_specs=pl.BlockSpec((rows_blk, dout), lambda i: (i, 0)),
        out_shape=jax.ShapeDtypeStruct((n, dout), jnp.float32),
    )(x, W, degcol)


def _fused_body(s_ref, p_ref, deg_ref, b_ref, w_ref, o_ref):
    d = lax.rsqrt(deg_ref[...] + 1.0)
    h = d * (s_ref[...] + p_ref[...]) + b_ref[...]
    h = jnp.maximum(h, 0.0)
    o_ref[...] = d * jnp.dot(h, w_ref[...], preferred_element_type=jnp.float32)


def _tc_fused_matmul(S, P, degcol, brow, W, rows_blk=2000):
    n, din = P.shape
    dout = W.shape[1]
    return pl.pallas_call(
        _fused_body,
        grid=(n // rows_blk,),
        in_specs=[
            pl.BlockSpec((rows_blk, din), lambda i: (i, 0)),
            pl.BlockSpec((rows_blk, din), lambda i: (i, 0)),
            pl.BlockSpec((rows_blk, 1), lambda i: (i, 0)),
            pl.BlockSpec((1, din), lambda i: (0, 0)),
            pl.BlockSpec((din, dout), lambda i: (0, 0)),
        ],
        out_specs=pl.BlockSpec((rows_blk, dout), lambda i: (i, 0)),
        out_shape=jax.ShapeDtypeStruct((n, dout), jnp.float32),
    )(S, P, degcol, brow, W)


def _final_body(s_ref, p_ref, deg_ref, b_ref, o_ref):
    d = lax.rsqrt(deg_ref[...] + 1.0)
    z = d * (s_ref[...] + p_ref[...]) + b_ref[...]
    m = jnp.max(z, axis=1, keepdims=True)
    lse = jnp.log(jnp.sum(jnp.exp(z - m), axis=1, keepdims=True)) + m
    o_ref[...] = z - lse


def _tc_final(S, P, degcol, brow, rows_blk=2000):
    n, d = P.shape
    return pl.pallas_call(
        _final_body,
        grid=(n // rows_blk,),
        in_specs=[
            pl.BlockSpec((rows_blk, d), lambda i: (i, 0)),
            pl.BlockSpec((rows_blk, d), lambda i: (i, 0)),
            pl.BlockSpec((rows_blk, 1), lambda i: (i, 0)),
            pl.BlockSpec((1, d), lambda i: (0, 0)),
        ],
        out_specs=pl.BlockSpec((rows_blk, d), lambda i: (i, 0)),
        out_shape=jax.ShapeDtypeStruct((n, d), jnp.float32),
    )(S, P, degcol, brow)


def kernel(x, edge_index, W1, b1, W2, b2, W3, b3, W4, b4):
    src = edge_index[0]
    dst = edge_index[1]

    deg = _sc_degree(dst).sum(axis=0)
    degcol = deg[:N, None]

    P1 = _tc_matmul(x, W1, degcol)
    S1 = _propagate[512](P1, src, dst)
    P2 = _tc_fused_matmul(S1, P1, degcol, b1[None, :], W2)
    S2 = _propagate[512](P2, src, dst)
    P3 = _tc_fused_matmul(S2, P2, degcol, b2[None, :], W3)
    S3 = _propagate[512](P3, src, dst)
    P4 = _tc_fused_matmul(S3, P3, degcol, b3[None, :], W4)
    S4 = _propagate[256](P4, src, dst)
    return _tc_final(S4, P4, degcol, b4[None, :])


# speedup vs baseline: 14.1434x; 1.0025x over previous
import functools

import jax
import jax.numpy as jnp
from jax import lax
from jax.experimental import pallas as pl
from jax.experimental.pallas import tpu as pltpu
from jax.experimental.pallas import tpu_sc as plsc

N = 10000
E = 160000
NPAD = 10240
NC = 2
NS = 16
NW = NC * NS
CH = 2000
NCH = E // CH
NV = CH // 16
QW = 128
BT = 80
EPT16 = E // NS
NBT = EPT16 // BT
RPS = NPAD // NS
BZ = 40

_mesh = lambda: plsc.VectorSubcoreMesh(
    core_axis_name="c", subcore_axis_name="s", num_cores=NC, num_subcores=NS)


NEQ = 4
NRG = NW // NEQ
RD = NPAD // NRG
EQ = E // NEQ
DCH = EQ // 2


def _deg_body(dst_hbm, out_hbm, dstb0, dstb1, cntf, cnt, sem0, sem1):
    wid = lax.axis_index("s") * NC + lax.axis_index("c")
    h = wid // NRG
    r = wid - h * NRG
    lo = r * RD
    e0 = h * EQ

    zf = jnp.zeros((16,), jnp.float32)
    ones = jnp.ones((16,), jnp.float32)
    lanes = lax.iota(jnp.int32, 16)

    bufs = (dstb0, dstb1)
    sems = (sem0, sem1)
    pltpu.async_copy(dst_hbm.at[pl.ds(e0, DCH)], dstb0, sem0)

    def zero(i, _):
        cntf[pl.ds(i * 16, 16)] = zf
        return 0

    lax.fori_loop(0, RD, zero, 0)

    def scan(dstb):
        def body(v, _):
            dv = dstb[pl.ds(v * 16, 16)]
            ld = dv - lo
            m = (ld >= 0) & (ld < RD)
            plsc.addupdate_scatter(cntf, [lanes * RD + ld], ones, mask=m)
            return 0
        lax.fori_loop(0, DCH // 16, body, 0, unroll=8)

    for c in range(2):
        if c + 1 < 2:
            pltpu.async_copy(dst_hbm.at[pl.ds(e0 + (c + 1) * DCH, DCH)],
                             bufs[(c + 1) % 2], sems[(c + 1) % 2])
        pltpu.make_async_copy(dst_hbm.at[pl.ds(e0 + c * DCH, DCH)],
                              bufs[c % 2], sems[c % 2]).wait()
        scan(bufs[c % 2])

    def red(g, _):
        t = cntf[pl.ds(g * 16, 16)]
        for l in range(1, 16):
            t = t + cntf[pl.ds(l * RD + g * 16, 16)]
        cnt[pl.ds(g * 16, 16)] = t
        return 0

    lax.fori_loop(0, RD // 16, red, 0)
    pltpu.sync_copy(cnt, out_hbm.at[h, pl.ds(lo, RD)])


def _sc_degree(dst):
    f = pl.kernel(
        _deg_body,
        out_type=jax.ShapeDtypeStruct((NEQ, NPAD), jnp.float32),
        mesh=_mesh(),
        compiler_params=pltpu.CompilerParams(needs_layout_passes=False),
        scratch_types=[
            pltpu.VMEM((DCH,), jnp.int32),
            pltpu.VMEM((DCH,), jnp.int32),
            pltpu.VMEM((16 * RD,), jnp.float32),
            pltpu.VMEM((RD,), jnp.float32),
            pltpu.SemaphoreType.DMA,
            pltpu.SemaphoreType.DMA,
        ],
    )
    return f(dst)


def _prop_body(p_hbm, src_hbm, dst_hbm, out_hbm,
               srcb, dstb, idxs0, idxs1, idxd0, idxd1, rows0, rows1, zbuf,
               shared, semg0, semg1, sems0, sems1, D):
    cid = lax.axis_index("c")
    sid = lax.axis_index("s")
    qpc = D // QW // NC

    zf = jnp.zeros((16,), jnp.float32)

    def zzero(i, _):
        r = i // (QW // 16)
        k = i - r * (QW // 16)
        zbuf[r, pl.ds(k * 16, 16)] = zf
        return 0

    base_e = sid * EPT16
    cps = pltpu.async_copy(src_hbm.at[pl.ds(base_e, EPT16)], srcb, semg0)
    cpd = pltpu.async_copy(dst_hbm.at[pl.ds(base_e, EPT16)], dstb, semg1)
    lax.fori_loop(0, BZ * (QW // 16), zzero, 0)
    cps.wait()
    cpd.wait()

    for qq in range(qpc):
        q = cid * qpc + qq
        c0 = pl.multiple_of(q * QW, QW)

        zcps = [pltpu.async_copy(
                    zbuf, shared.at[pl.ds(sid * RPS + z * BZ, BZ)], semg0)
                for z in range(RPS // BZ)]
        for cp in zcps:
            cp.wait()
        plsc.subcore_barrier()

        def start_gather(b, idxs, rows, semg):
            for g in range(BT // 16):
                idxs[pl.ds(g * 16, 16)] = srcb[pl.ds(b * BT + g * 16, 16)]
            return pltpu.async_copy(p_hbm.at[idxs, pl.ds(c0, QW)], rows, semg)

        start_gather(0, idxs0, rows0, semg0)

        def half(b, idxs_n, rows_n, semg_n, idxd_n, sems_n,
                 rows_c, semg_c, idxd_c, sems_c):
            @pl.when(b < NBT)
            def _():
                @pl.when(b >= 1)
                def _():
                    pltpu.make_async_copy(
                        rows_n, shared.at[idxd_n], sems_n).wait()

                @pl.when(b + 1 < NBT)
                def _():
                    start_gather(b + 1, idxs_n, rows_n, semg_n)
                pltpu.make_async_copy(
                    p_hbm.at[idxs_n, pl.ds(c0, QW)], rows_c, semg_c).wait()
                for g in range(BT // 16):
                    idxd_c[pl.ds(g * 16, 16)] = dstb[pl.ds(b * BT + g * 16, 16)]
                pltpu.async_copy(rows_c, shared.at[idxd_c], sems_c, add=True)
            return 0

        def pair(i, _):
            b = i * 2
            half(b, idxs1, rows1, semg1, idxd1, sems1,
                 rows0, semg0, idxd0, sems0)
            half(b + 1, idxs0, rows0, semg0, idxd0, sems0,
                 rows1, semg1, idxd1, sems1)
            return 0

        lax.fori_loop(0, (NBT + 1) // 2, pair, 0)
        lastp = (NBT - 1) % 2
        pltpu.make_async_copy(
            (rows0, rows1)[lastp],
            shared.at[(idxd0, idxd1)[lastp]],
            (sems0, sems1)[lastp]).wait()
        plsc.subcore_barrier()
        pltpu.sync_copy(shared.at[pl.ds(sid * RPS, RPS)],
                        out_hbm.at[pl.ds(sid * RPS, RPS), pl.ds(c0, QW)])
        plsc.subcore_barrier()


def _make_propagate(D):
    body = functools.partial(_prop_body, D=D)
    return pl.kernel(
        body,
        out_type=jax.ShapeDtypeStruct((NPAD, D), jnp.float32),
        mesh=_mesh(),
        compiler_params=pltpu.CompilerParams(needs_layout_passes=False),
        scratch_types=[
            pltpu.VMEM((EPT16,), jnp.int32),
            pltpu.VMEM((EPT16,), jnp.int32),
            pltpu.VMEM((BT,), jnp.int32),
            pltpu.VMEM((BT,), jnp.int32),
            pltpu.VMEM((BT,), jnp.int32),
            pltpu.VMEM((BT,), jnp.int32),
            pltpu.VMEM((BT, QW), jnp.float32),
            pltpu.VMEM((BT, QW), jnp.float32),
            pltpu.VMEM((BZ, QW), jnp.float32),
            pltpu.VMEM_SHARED((NPAD, QW), jnp.float32),
            pltpu.SemaphoreType.DMA,
            pltpu.SemaphoreType.DMA,
            pltpu.SemaphoreType.DMA,
            pltpu.SemaphoreType.DMA,
        ],
    )


_propagate = {D: _make_propagate(D) for D in (512, 256)}


def _mm_body(x_ref, w_ref, deg_ref, o_ref):
    d = lax.rsqrt(deg_ref[...] + 1.0)
    o_ref[...] = d * jnp.dot(x_ref[...], w_ref[...],
                             preferred_element_type=jnp.float32)


def _tc_matmul(x, W, degcol, rows_blk=2000):
    n, din = x.shape
    dout = W.shape[1]
    return pl.pallas_call(
        _mm_body,
        grid=(n // rows_blk,),
        in_specs=[
            pl.BlockSpec((rows_blk, din), lambda i: (i, 0)),
            pl.BlockSpec((din, dout), lambda i: (0, 0)),
            pl.BlockSpec((rows_blk, 1), lambda i: (i, 0)),
        ],
        out_specs=pl.BlockSpec((rows_blk, dout), lambda i: (i, 0)),
        out_shape=jax.ShapeDtypeStruct((n, dout), jnp.float32),
    )(x, W, degcol)


def _fused_body(s_ref, p_ref, deg_ref, b_ref, w_ref, o_ref):
    d = lax.rsqrt(deg_ref[...] + 1.0)
    h = d * (s_ref[...] + p_ref[...]) + b_ref[...]
    h = jnp.maximum(h, 0.0)
    o_ref[...] = d * jnp.dot(h, w_ref[...], preferred_element_type=jnp.float32)


def _tc_fused_matmul(S, P, degcol, brow, W, rows_blk=2000):
    n, din = P.shape
    dout = W.shape[1]
    return pl.pallas_call(
        _fused_body,
        grid=(n // rows_blk,),
        in_specs=[
            pl.BlockSpec((rows_blk, din), lambda i: (i, 0)),
            pl.BlockSpec((rows_blk, din), lambda i: (i, 0)),
            pl.BlockSpec((rows_blk, 1), lambda i: (i, 0)),
            pl.BlockSpec((1, din), lambda i: (0, 0)),
            pl.BlockSpec((din, dout), lambda i: (0, 0)),
        ],
        out_specs=pl.BlockSpec((rows_blk, dout), lambda i: (i, 0)),
        out_shape=jax.ShapeDtypeStruct((n, dout), jnp.float32),
    )(S, P, degcol, brow, W)


def _final_body(s_ref, p_ref, deg_ref, b_ref, o_ref):
    d = lax.rsqrt(deg_ref[...] + 1.0)
    z = d * (s_ref[...] + p_ref[...]) + b_ref[...]
    m = jnp.max(z, axis=1, keepdims=True)
    lse = jnp.log(jnp.sum(jnp.exp(z - m), axis=1, keepdims=True)) + m
    o_ref[...] = z - lse


def _tc_final(S, P, degcol, brow, rows_blk=2000):
    n, d = P.shape
    return pl.pallas_call(
        _final_body,
        grid=(n // rows_blk,),
        in_specs=[
            pl.BlockSpec((rows_blk, d), lambda i: (i, 0)),
            pl.BlockSpec((rows_blk, d), lambda i: (i, 0)),
            pl.BlockSpec((rows_blk, 1), lambda i: (i, 0)),
            pl.BlockSpec((1, d), lambda i: (0, 0)),
        ],
        out_specs=pl.BlockSpec((rows_blk, d), lambda i: (i, 0)),
        out_shape=jax.ShapeDtypeStruct((n, d), jnp.float32),
    )(S, P, degcol, brow)


def kernel(x, edge_index, W1, b1, W2, b2, W3, b3, W4, b4):
    src = edge_index[0]
    dst = edge_index[1]

    deg = _sc_degree(dst).sum(axis=0)
    degcol = deg[:N, None]

    P1 = _tc_matmul(x, W1, degcol)
    S1 = _propagate[512](P1, src, dst)
    P2 = _tc_fused_matmul(S1, P1, degcol, b1[None, :], W2)
    S2 = _propagate[512](P2, src, dst)
    P3 = _tc_fused_matmul(S2, P2, degcol, b2[None, :], W3)
    S3 = _propagate[512](P3, src, dst)
    P4 = _tc_fused_matmul(S3, P3, degcol, b3[None, :], W4)
    S4 = _propagate[256](P4, src, dst)
    return _tc_final(S4, P4, degcol, b4[None, :])
